# baseline (speedup 1.0000x reference)
import jax
import jax.numpy as jnp
from jax import lax
from jax.experimental import pallas as pl

N_TOKENS = 1000
ROWS = 64


def _onehot_block(x_ref, o_ref):
    xb = x_ref[...]
    iota = lax.broadcasted_iota(jnp.int32, (ROWS, x_ref.shape[1], N_TOKENS), 2)
    o_ref[...] = (iota == xb).astype(o_ref.dtype)


def kernel(x):
    B, T = x.shape
    x3 = x[:, :, None]
    out = pl.pallas_call(
        _onehot_block,
        grid=(B // ROWS,),
        in_specs=[pl.BlockSpec((ROWS, T, 1), lambda i: (i, 0, 0))],
        out_specs=pl.BlockSpec((ROWS, T, N_TOKENS), lambda i: (i, 0, 0)),
        out_shape=jax.ShapeDtypeStruct((B, T, N_TOKENS), x.dtype),
    )(x3)
    return out
